# baseline (speedup 1.0000x reference)
import functools
import math

import jax
import jax.numpy as jnp
from jax import lax
from jax.experimental import pallas as pl
from jax.experimental.pallas import tpu as pltpu
from jax.experimental.pallas import tpu_sc as plsc

NC = 2
NS = 16
L = 16

C = 128


def _rsqrt(a):
    i = lax.bitcast_convert_type(a, jnp.int32)
    i = 0x5F3759DF - lax.shift_right_logical(i, 1)
    y = lax.bitcast_convert_type(i, jnp.float32)
    for _ in range(3):
        y = y * (1.5 - 0.5 * a * y * y)
    return y


def _make_sc_kernel(N, V, P, D):
    NW = NC * NS
    per_w = N // NW
    nch = per_w // C
    groups = C // L
    scale = math.sqrt(D)
    mesh = plsc.VectorSubcoreMesh(core_axis_name="c", subcore_axis_name="s")

    @functools.partial(
        pl.kernel,
        mesh=mesh,
        compiler_params=pltpu.CompilerParams(
            needs_layout_passes=False, use_tc_tiling_on_sc=False),
        out_type=jax.ShapeDtypeStruct((N, D), jnp.float32),
        scratch_types=[
            pltpu.VMEM((C,), jnp.int32),
            pltpu.VMEM((C,), jnp.int32),
            pltpu.VMEM((C, D), jnp.float32),
            pltpu.VMEM((P * D,), jnp.float32),
            pltpu.VMEM((D,), jnp.float32),
            pltpu.VMEM((D,), jnp.float32),
            pltpu.VMEM((D, L), jnp.float32),
            pltpu.SemaphoreType.DMA,
        ],
    )
    def sc_kernel(idx_hbm, pid_hbm, item_hbm, pos_hbm, gam_hbm, bet_hbm,
                  out_hbm, idx_v, pid_v, rows_v, pos_v, gam_v, bet_v, xT, sem):
        wid = lax.axis_index("s") * NC + lax.axis_index("c")
        base = wid * per_w

        pltpu.sync_copy(pos_hbm, pos_v)
        pltpu.sync_copy(gam_hbm, gam_v)
        pltpu.sync_copy(bet_hbm, bet_v)

        lanes = lax.broadcasted_iota(jnp.int32, (L,), 0)
        gvec = [gam_v[pl.ds(k * L, L)] for k in range(D // L)]
        bvec = [bet_v[pl.ds(k * L, L)] for k in range(D // L)]

        def chunk_body(ci, _):
            row0 = base + ci * C
            pltpu.sync_copy(idx_hbm.at[pl.ds(row0, C)], idx_v)
            pltpu.sync_copy(pid_hbm.at[pl.ds(row0, C)], pid_v)
            pltpu.async_copy(item_hbm.at[idx_v], rows_v, sem).wait()

            def group_body(gi, _):
                g0 = gi * L
                ridx = g0 + lanes
                pid16 = pid_v[pl.ds(g0, L)]
                pbase = pid16 * D

                def col1(j, carry):
                    s, q = carry
                    cj = jnp.full((L,), j, jnp.int32)
                    iv = plsc.load_gather(rows_v, [ridx, cj])
                    pv = plsc.load_gather(pos_v, [pbase + j])
                    x = iv * scale + pv
                    xT[j] = x
                    return s + x, q + x * x

                zero = jnp.zeros((L,), jnp.float32)
                s, q = lax.fori_loop(0, D, col1, (zero, zero))
                mu = s * (1.0 / D)
                var = q * (1.0 / D) - mu * mu
                rstd = _rsqrt(var + 1e-5)

                def col2(j, _):
                    cj = jnp.full((L,), j, jnp.int32)
                    xh = (xT[j] - mu) * rstd
                    plsc.store_scatter(rows_v, [ridx, cj], xh)
                    return 0

                lax.fori_loop(0, D, col2, 0)

                def row3(r, _):
                    rr = g0 + r
                    for k in range(D // L):
                        v = rows_v[rr, pl.ds(k * L, L)]
                        rows_v[rr, pl.ds(k * L, L)] = v * gvec[k] + bvec[k]
                    return 0

                lax.fori_loop(0, L, row3, 0)
                return 0

            lax.fori_loop(0, groups, group_body, 0)
            pltpu.sync_copy(rows_v, out_hbm.at[pl.ds(row0, C)])
            return 0

        lax.fori_loop(0, nch, chunk_body, 0)

    return sc_kernel


def kernel(input_sequence, position_ids, item_table, pos_table, ln_gamma, ln_beta):
    I, B = input_sequence.shape
    V, D = item_table.shape
    P = pos_table.shape[0]
    N = I * B
    sc = _make_sc_kernel(N, V, P, D)
    out = sc(
        input_sequence.reshape(N),
        position_ids.reshape(N),
        item_table,
        pos_table.reshape(P * D),
        ln_gamma,
        ln_beta,
    )
    return out.reshape(I, B, D)


# speedup vs baseline: 1.0712x; 1.0712x over previous
import functools
import math

import jax
import jax.numpy as jnp
from jax import lax
from jax.experimental import pallas as pl
from jax.experimental.pallas import tpu as pltpu
from jax.experimental.pallas import tpu_sc as plsc

NC = 2
NS = 16
L = 16

C = 256


def _rsqrt(a):
    i = lax.bitcast_convert_type(a, jnp.int32)
    i = 0x5F3759DF - lax.shift_right_logical(i, 1)
    y = lax.bitcast_convert_type(i, jnp.float32)
    for _ in range(3):
        y = y * (1.5 - 0.5 * a * y * y)
    return y


def _make_sc_kernel(N, V, P, D):
    NW = NC * NS
    per_w = N // NW
    nch = per_w // C
    groups = C // L
    scale = math.sqrt(D)
    mesh = plsc.VectorSubcoreMesh(core_axis_name="c", subcore_axis_name="s")

    @functools.partial(
        pl.kernel,
        mesh=mesh,
        compiler_params=pltpu.CompilerParams(
            needs_layout_passes=False, use_tc_tiling_on_sc=False),
        out_type=jax.ShapeDtypeStruct((N, D), jnp.float32),
        scratch_types=[
            pltpu.VMEM((C,), jnp.int32),
            pltpu.VMEM((C,), jnp.int32),
            pltpu.VMEM((C,), jnp.int32),
            pltpu.VMEM((C,), jnp.int32),
            pltpu.VMEM((C, D), jnp.float32),
            pltpu.VMEM((C, D), jnp.float32),
            pltpu.VMEM((C, D), jnp.float32),
            pltpu.VMEM((C, D), jnp.float32),
            pltpu.VMEM((P * D,), jnp.float32),
            pltpu.VMEM((D,), jnp.float32),
            pltpu.VMEM((D,), jnp.float32),
            pltpu.VMEM((D, L), jnp.float32),
            pltpu.SemaphoreType.DMA,
            pltpu.SemaphoreType.DMA,
            pltpu.SemaphoreType.DMA,
            pltpu.SemaphoreType.DMA,
        ],
    )
    def sc_kernel(idx_hbm, pid_hbm, item_hbm, pos_hbm, gam_hbm, bet_hbm,
                  out_hbm, idx0, idx1, pid0, pid1, rows0, rows1, ob0, ob1,
                  pos_v, gam_v, bet_v, xT, sg0, sg1, so0, so1):
        idxs, pids, rows, obs = [idx0, idx1], [pid0, pid1], [rows0, rows1], [ob0, ob1]
        sgs, sos = [sg0, sg1], [so0, so1]
        wid = lax.axis_index("s") * NC + lax.axis_index("c")
        base = wid * per_w

        pltpu.sync_copy(pos_hbm, pos_v)
        pltpu.sync_copy(gam_hbm, gam_v)
        pltpu.sync_copy(bet_hbm, bet_v)

        lanes = lax.broadcasted_iota(jnp.int32, (L,), 0)
        gvec = [gam_v[pl.ds(k * L, L)] for k in range(D // L)]
        bvec = [bet_v[pl.ds(k * L, L)] for k in range(D // L)]

        def fire_in(j, s):
            row0 = base + j * C
            pltpu.sync_copy(idx_hbm.at[pl.ds(row0, C)], idxs[s])
            pltpu.sync_copy(pid_hbm.at[pl.ds(row0, C)], pids[s])
            pltpu.async_copy(item_hbm.at[idxs[s]], rows[s], sgs[s])

        def compute(s):
            rv, pv_ids, ob = rows[s], pids[s], obs[s]

            def group_body(gi, _):
                g0 = gi * L
                ridx = g0 + lanes
                pid16 = pv_ids[pl.ds(g0, L)]
                pbase = pid16 * D

                def col1(j, carry):
                    sacc, qacc = carry
                    cj = jnp.full((L,), j, jnp.int32)
                    iv = plsc.load_gather(rv, [ridx, cj])
                    pvv = plsc.load_gather(pos_v, [pbase + j])
                    x = iv * scale + pvv
                    xT[j] = x
                    return sacc + x, qacc + x * x

                zero = jnp.zeros((L,), jnp.float32)
                sacc, qacc = lax.fori_loop(0, D, col1, (zero, zero))
                mu = sacc * (1.0 / D)
                var = qacc * (1.0 / D) - mu * mu
                rstd = _rsqrt(var + 1e-5)

                def col2(j, _):
                    cj = jnp.full((L,), j, jnp.int32)
                    xh = (xT[j] - mu) * rstd
                    plsc.store_scatter(ob, [ridx, cj], xh)
                    return 0

                lax.fori_loop(0, D, col2, 0)

                def row3(r, _):
                    rr = g0 + r
                    for k in range(D // L):
                        v = ob[rr, pl.ds(k * L, L)]
                        ob[rr, pl.ds(k * L, L)] = v * gvec[k] + bvec[k]
                    return 0

                lax.fori_loop(0, L, row3, 0)
                return 0

            lax.fori_loop(0, groups, group_body, 0)

        fire_in(0, 0)

        def pair_body(ci, _):
            for b in range(2):
                j = 2 * ci + b

                @pl.when(j + 1 < nch)
                def _():
                    fire_in(j + 1, 1 - b)

                pltpu.make_async_copy(
                    item_hbm.at[idxs[b]], rows[b], sgs[b]).wait()

                @pl.when(j >= 2)
                def _():
                    pltpu.make_async_copy(
                        obs[b], out_hbm.at[pl.ds(base, C)], sos[b]).wait()

                compute(b)
                pltpu.async_copy(
                    obs[b], out_hbm.at[pl.ds(base + j * C, C)], sos[b])
            return 0

        lax.fori_loop(0, nch // 2, pair_body, 0)
        pltpu.make_async_copy(obs[0], out_hbm.at[pl.ds(base, C)], sos[0]).wait()
        pltpu.make_async_copy(obs[1], out_hbm.at[pl.ds(base, C)], sos[1]).wait()

    return sc_kernel


def kernel(input_sequence, position_ids, item_table, pos_table, ln_gamma, ln_beta):
    I, B = input_sequence.shape
    V, D = item_table.shape
    P = pos_table.shape[0]
    N = I * B
    sc = _make_sc_kernel(N, V, P, D)
    out = sc(
        input_sequence.reshape(N),
        position_ids.reshape(N),
        item_table,
        pos_table.reshape(P * D),
        ln_gamma,
        ln_beta,
    )
    return out.reshape(I, B, D)
